# Initial kernel scaffold; baseline (speedup 1.0000x reference)
import jax
import jax.numpy as jnp
from jax.experimental import pallas as pl


def kernel(q, paged_kv_cache, kv_page_indptr, kv_page_indices, sparse_ind, sparse_nnz):
    raise NotImplementedError("write your pallas kernel here")


# speedup vs baseline: 1.8304x; 1.8304x over previous
import functools
import math

import jax
import jax.numpy as jnp
from jax import lax
from jax.experimental import pallas as pl
from jax.experimental.pallas import tpu as pltpu
from jax.experimental.pallas import tpu_sc as plsc

_NC = 2
_NS = 16
_LANES = 16

_TOKENS = 2048


def _sc_counts(npairs: int, lmax: int):
    slot = _TOKENS + 8
    nw = _NC * _NS
    per_w = npairs // nw

    mesh = plsc.VectorSubcoreMesh(
        core_axis_name="c", subcore_axis_name="s",
        num_cores=_NC, num_subcores=_NS)

    @functools.partial(
        pl.kernel,
        out_type=jax.ShapeDtypeStruct((npairs, _TOKENS), jnp.float32),
        mesh=mesh,
        scratch_types=[
            pltpu.VMEM((lmax,), jnp.int32),
            pltpu.VMEM((lmax,), jnp.int32),
            pltpu.VMEM((lmax,), jnp.float32),
            pltpu.VMEM((slot,), jnp.float32),
            pltpu.VMEM((_LANES,), jnp.int32),
            pltpu.VMEM_SHARED((_NS * slot,), jnp.float32),
        ],
        compiler_params=pltpu.CompilerParams(use_tc_tiling_on_sc=False),
    )
    def counts_kernel(ind_hbm, nnz_hbm, out_hbm, ind_v, idx_v, ones_v, zeros_v,
                      nnz16_v, acc_sh):
        cid = lax.axis_index("c")
        sid = lax.axis_index("s")
        wid = sid * _NC + cid
        base_slot = sid * slot

        for i in range(lmax // _LANES):
            ones_v[pl.ds(i * _LANES, _LANES)] = jnp.full((_LANES,), 1.0, jnp.float32)
        for i in range(slot // _LANES):
            zeros_v[pl.ds(i * _LANES, _LANES)] = jnp.zeros((_LANES,), jnp.float32)
        for p in range(per_w):
            pair = wid * per_w + p
            pltpu.sync_copy(zeros_v, acc_sh.at[pl.ds(base_slot, slot)])
            pltpu.sync_copy(ind_hbm.at[pair], ind_v)
            pltpu.sync_copy(nnz_hbm.at[lax.full((_LANES,), pair, jnp.int32)],
                            nnz16_v)
            nnz_splat = nnz16_v[...]
            for ci in range(lmax // _LANES):
                lvec = lax.iota(jnp.int32, _LANES) + ci * _LANES
                t = ind_v[pl.ds(ci * _LANES, _LANES)]
                ok = lvec < nnz_splat
                idx_v[pl.ds(ci * _LANES, _LANES)] = (
                    jnp.where(ok, t, _TOKENS) + base_slot)
            pltpu.sync_copy(ones_v, acc_sh.at[idx_v], add=True)
            pltpu.sync_copy(acc_sh.at[pl.ds(base_slot, _TOKENS)],
                            out_hbm.at[pair])

    return counts_kernel


def _tc_attention(B, H, Hkv, page_size, D, interpret=False):
    G = H // Hkv
    npages = _TOKENS // page_size
    scale = 1.0 / math.sqrt(D)

    def body(indptr_s, pidx_s, kv_ref, q_ref, c_ref, out_ref, k_scr, v_scr):
        j = pl.program_id(2)

        @pl.when(j < npages)
        def _stage():
            off = j * page_size
            k_scr[pl.ds(off, page_size), :] = kv_ref[0, 0, 0]
            v_scr[pl.ds(off, page_size), :] = kv_ref[0, 1, 0]

        @pl.when(j == npages)
        def _finish():
            qg = q_ref[0, :, 0, :]
            c = c_ref[0]
            fs = lax.dot_general(
                qg, k_scr[...], (((1,), (1,)), ((), ())),
                preferred_element_type=jnp.float32) * scale
            fsm = jnp.where(c > 0.0, fs, -1e30)
            m = jnp.max(fsm, axis=1, keepdims=True)
            p = c * jnp.exp(fsm - m)
            denom = jnp.sum(p, axis=1, keepdims=True)
            w = p / jnp.maximum(denom, 1e-30)
            out_ref[0, :, 0, :] = lax.dot_general(
                w, v_scr[...], (((1,), (0,)), ((), ())),
                preferred_element_type=jnp.float32)

    def kv_index(b, hk, j, indptr, pidx):
        jc = jnp.minimum(j, npages - 1)
        return (pidx[indptr[b] + jc], 0, hk, 0, 0)

    grid_spec = pltpu.PrefetchScalarGridSpec(
        num_scalar_prefetch=2,
        grid=(B, Hkv, npages + 1),
        in_specs=[
            pl.BlockSpec((1, 2, 1, page_size, D), kv_index),
            pl.BlockSpec((1, G, 1, D), lambda b, hk, j, *_: (b, hk, 0, 0)),
            pl.BlockSpec((1, G, _TOKENS),
                         lambda b, hk, j, *_: (b * Hkv + hk, 0, 0)),
        ],
        out_specs=pl.BlockSpec((1, G, 1, D), lambda b, hk, j, *_: (b, hk, 0, 0)),
        scratch_shapes=[
            pltpu.VMEM((_TOKENS, D), jnp.float32),
            pltpu.VMEM((_TOKENS, D), jnp.float32),
        ],
    )
    return pl.pallas_call(
        body,
        grid_spec=grid_spec,
        out_shape=jax.ShapeDtypeStruct((B, H, 1, D), jnp.float32),
        compiler_params=pltpu.CompilerParams(
            dimension_semantics=("arbitrary", "arbitrary", "arbitrary")),
        interpret=interpret,
    )


def kernel(q, paged_kv_cache, kv_page_indptr, kv_page_indices, sparse_ind,
           sparse_nnz):
    B, H, _, D = q.shape
    _, _, Hkv, page_size, _ = paged_kv_cache.shape
    Lmax = sparse_ind.shape[2]
    npairs = B * H

    ind_flat = sparse_ind.reshape(npairs, Lmax)
    nnz_flat = sparse_nnz.reshape(npairs)

    counts = _sc_counts(npairs, Lmax)(ind_flat, nnz_flat)
    counts3 = counts.reshape(B * Hkv, H // Hkv, _TOKENS)

    return _tc_attention(B, H, Hkv, page_size, D)(
        kv_page_indptr, kv_page_indices, paged_kv_cache, q, counts3)


# speedup vs baseline: 31.1861x; 17.0374x over previous
import functools
import math

import jax
import jax.numpy as jnp
from jax import lax
from jax.experimental import pallas as pl
from jax.experimental.pallas import tpu as pltpu
from jax.experimental.pallas import tpu_sc as plsc

_NC = 2
_NS = 16
_LANES = 16

_TOKENS = 2048


def _sc_counts(npairs: int, lmax: int):
    slot = _TOKENS + 8
    nw = _NC * _NS
    per_w = npairs // nw

    mesh = plsc.VectorSubcoreMesh(
        core_axis_name="c", subcore_axis_name="s",
        num_cores=_NC, num_subcores=_NS)

    @functools.partial(
        pl.kernel,
        out_type=jax.ShapeDtypeStruct((npairs, _TOKENS), jnp.float32),
        mesh=mesh,
        scratch_types=[
            pltpu.VMEM((lmax,), jnp.int32),
            pltpu.VMEM((lmax,), jnp.int32),
            pltpu.VMEM((lmax,), jnp.float32),
            pltpu.VMEM((slot,), jnp.float32),
            pltpu.VMEM((_LANES,), jnp.int32),
            pltpu.VMEM_SHARED((_NS * slot,), jnp.float32),
        ],
        compiler_params=pltpu.CompilerParams(use_tc_tiling_on_sc=False),
    )
    def counts_kernel(ind_hbm, nnz_hbm, out_hbm, ind_v, idx_v, ones_v, zeros_v,
                      nnz16_v, acc_sh):
        cid = lax.axis_index("c")
        sid = lax.axis_index("s")
        wid = sid * _NC + cid
        base_slot = sid * slot

        for i in range(lmax // _LANES):
            ones_v[pl.ds(i * _LANES, _LANES)] = jnp.full((_LANES,), 1.0, jnp.float32)
        for i in range(slot // _LANES):
            zeros_v[pl.ds(i * _LANES, _LANES)] = jnp.zeros((_LANES,), jnp.float32)
        for p in range(per_w):
            pair = wid * per_w + p
            pltpu.sync_copy(zeros_v, acc_sh.at[pl.ds(base_slot, slot)])
            pltpu.sync_copy(ind_hbm.at[pair], ind_v)
            pltpu.sync_copy(nnz_hbm.at[lax.full((_LANES,), pair, jnp.int32)],
                            nnz16_v)
            nnz_splat = nnz16_v[...]
            for ci in range(lmax // _LANES):
                lvec = lax.iota(jnp.int32, _LANES) + ci * _LANES
                t = ind_v[pl.ds(ci * _LANES, _LANES)]
                ok = lvec < nnz_splat
                idx_v[pl.ds(ci * _LANES, _LANES)] = (
                    jnp.where(ok, t, _TOKENS) + base_slot)
            pltpu.sync_copy(ones_v, acc_sh.at[idx_v], add=True)
            pltpu.sync_copy(acc_sh.at[pl.ds(base_slot, _TOKENS)],
                            out_hbm.at[pair])

    return counts_kernel


def _tc_attention(B, H, Hkv, page_size, D, interpret=False):
    G = H // Hkv
    npages = _TOKENS // page_size
    scale = 1.0 / math.sqrt(D)
    total = B * Hkv

    def body(indptr_s, pidx_s, cache, q_ref, c_ref, out_ref, kv_scr, sems):
        b = pl.program_id(0)
        hk = pl.program_id(1)
        step = b * Hkv + hk
        ns = step + 1
        nb = ns // Hkv
        nh = lax.rem(ns, Hkv)

        def issue(dstbuf, bb, hh):
            base = indptr_s[bb]

            def one(j, _):
                pid = pidx_s[base + j]
                pltpu.make_async_copy(
                    cache.at[pid, :, hh],
                    kv_scr.at[dstbuf, :, pl.ds(j * page_size, page_size), :],
                    sems.at[dstbuf]).start()
                return 0

            lax.fori_loop(0, npages, one, 0)

        @pl.when(step == 0)
        def _prologue():
            issue(0, b, hk)

        def phase(cur):
            @pl.when(ns < total)
            def _():
                issue(1 - cur, nb, nh)

            def wone(j, _):
                pltpu.make_async_copy(
                    cache.at[0, :, 0],
                    kv_scr.at[cur, :, pl.ds(0, page_size), :],
                    sems.at[cur]).wait()
                return 0

            lax.fori_loop(0, npages, wone, 0)

            k_all = kv_scr[cur, 0]
            v_all = kv_scr[cur, 1]
            qg = q_ref[0, :, 0, :]
            c = c_ref[0]
            fs = lax.dot_general(
                qg, k_all, (((1,), (1,)), ((), ())),
                preferred_element_type=jnp.float32) * scale
            fsm = jnp.where(c > 0.0, fs, -1e30)
            m = jnp.max(fsm, axis=1, keepdims=True)
            p = c * jnp.exp(fsm - m)
            denom = jnp.sum(p, axis=1, keepdims=True)
            w = p / jnp.maximum(denom, 1e-30)
            out_ref[0, :, 0, :] = lax.dot_general(
                w, v_all, (((1,), (0,)), ((), ())),
                preferred_element_type=jnp.float32)

        par = lax.rem(step, 2)

        @pl.when(par == 0)
        def _even():
            phase(0)

        @pl.when(par == 1)
        def _odd():
            phase(1)

    grid_spec = pltpu.PrefetchScalarGridSpec(
        num_scalar_prefetch=2,
        grid=(B, Hkv),
        in_specs=[
            pl.BlockSpec(memory_space=pl.ANY),
            pl.BlockSpec((1, G, 1, D), lambda b, hk, *_: (b, hk, 0, 0)),
            pl.BlockSpec((1, G, _TOKENS), lambda b, hk, *_: (b * Hkv + hk, 0, 0)),
        ],
        out_specs=pl.BlockSpec((1, G, 1, D), lambda b, hk, *_: (b, hk, 0, 0)),
        scratch_shapes=[
            pltpu.VMEM((2, 2, _TOKENS, D), jnp.float32),
            pltpu.SemaphoreType.DMA((2,)),
        ],
    )
    return pl.pallas_call(
        body,
        grid_spec=grid_spec,
        out_shape=jax.ShapeDtypeStruct((B, H, 1, D), jnp.float32),
        compiler_params=pltpu.CompilerParams(
            dimension_semantics=("arbitrary", "arbitrary")),
        interpret=interpret,
    )


def kernel(q, paged_kv_cache, kv_page_indptr, kv_page_indices, sparse_ind,
           sparse_nnz):
    B, H, _, D = q.shape
    _, _, Hkv, page_size, _ = paged_kv_cache.shape
    Lmax = sparse_ind.shape[2]
    npairs = B * H

    ind_flat = sparse_ind.reshape(npairs, Lmax)
    nnz_flat = sparse_nnz.reshape(npairs)

    counts = _sc_counts(npairs, Lmax)(ind_flat, nnz_flat)
    counts3 = counts.reshape(B * Hkv, H // Hkv, _TOKENS)

    return _tc_attention(B, H, Hkv, page_size, D)(
        kv_page_indptr, kv_page_indices, paged_kv_cache, q, counts3)


# speedup vs baseline: 64.2284x; 2.0595x over previous
import functools
import math

import jax
import jax.numpy as jnp
from jax import lax
from jax.experimental import pallas as pl
from jax.experimental.pallas import tpu as pltpu
from jax.experimental.pallas import tpu_sc as plsc

_NC = 2
_NS = 16
_LANES = 16

_TOKENS = 2048


def _sc_counts(npairs: int, lmax: int):
    slot = _TOKENS + 8
    nw = _NC * _NS
    per_w = npairs // nw

    mesh = plsc.VectorSubcoreMesh(
        core_axis_name="c", subcore_axis_name="s",
        num_cores=_NC, num_subcores=_NS)

    @functools.partial(
        pl.kernel,
        out_type=jax.ShapeDtypeStruct((npairs, _TOKENS), jnp.float32),
        mesh=mesh,
        scratch_types=[
            pltpu.VMEM((lmax,), jnp.int32),
            pltpu.VMEM((lmax,), jnp.int32),
            pltpu.VMEM((lmax,), jnp.float32),
            pltpu.VMEM((slot,), jnp.float32),
            pltpu.VMEM((_LANES,), jnp.int32),
            pltpu.VMEM_SHARED((_NS * slot,), jnp.float32),
        ],
        compiler_params=pltpu.CompilerParams(use_tc_tiling_on_sc=False),
    )
    def counts_kernel(ind_hbm, nnz_hbm, out_hbm, ind_v, idx_v, ones_v, zeros_v,
                      nnz16_v, acc_sh):
        cid = lax.axis_index("c")
        sid = lax.axis_index("s")
        wid = sid * _NC + cid
        base_slot = sid * slot

        for i in range(lmax // _LANES):
            ones_v[pl.ds(i * _LANES, _LANES)] = jnp.full((_LANES,), 1.0, jnp.float32)
        for i in range(slot // _LANES):
            zeros_v[pl.ds(i * _LANES, _LANES)] = jnp.zeros((_LANES,), jnp.float32)
        for p in range(per_w):
            pair = wid * per_w + p
            pltpu.sync_copy(zeros_v, acc_sh.at[pl.ds(base_slot, slot)])
            pltpu.sync_copy(ind_hbm.at[pair], ind_v)
            pltpu.sync_copy(nnz_hbm.at[lax.full((_LANES,), pair, jnp.int32)],
                            nnz16_v)
            nnz_splat = nnz16_v[...]
            for ci in range(lmax // _LANES):
                lvec = lax.iota(jnp.int32, _LANES) + ci * _LANES
                t = ind_v[pl.ds(ci * _LANES, _LANES)]
                ok = lvec < nnz_splat
                idx_v[pl.ds(ci * _LANES, _LANES)] = (
                    jnp.where(ok, t, _TOKENS) + base_slot)
            pltpu.sync_copy(ones_v, acc_sh.at[idx_v], add=True)
            pltpu.sync_copy(acc_sh.at[pl.ds(base_slot, _TOKENS)],
                            out_hbm.at[pair])

    return counts_kernel


def _tc_attention(B, H, Hkv, page_size, D, interpret=False):
    G = H // Hkv
    npages = _TOKENS // page_size
    scale = 1.0 / math.sqrt(D)
    total = B * Hkv

    def body(indptr_s, pidx_s, cache, q_ref, c_ref, out_ref, kv_scr, sems):
        b = pl.program_id(0)

        def issue(dstbuf, bb):
            base = indptr_s[bb]

            def one(j, _):
                pid = pidx_s[base + j]
                pltpu.make_async_copy(
                    cache.at[pid],
                    kv_scr.at[dstbuf, :, :, j],
                    sems.at[dstbuf]).start()
                return 0

            lax.fori_loop(0, npages, one, 0)

        @pl.when(b == 0)
        def _prologue():
            issue(0, b)

        def phase(cur):
            @pl.when(b + 1 < B)
            def _():
                issue(1 - cur, b + 1)

            def wone(j, _):
                pltpu.make_async_copy(
                    cache.at[0],
                    kv_scr.at[cur, :, :, 0],
                    sems.at[cur]).wait()
                return 0

            lax.fori_loop(0, npages, wone, 0)

            qv = q_ref[0, :, 0, :]
            cv = c_ref[0]
            for hk in range(Hkv):
                k_all = kv_scr[cur, 0, hk].reshape(_TOKENS, D)
                v_all = kv_scr[cur, 1, hk].reshape(_TOKENS, D)
                qg = qv[hk * G:(hk + 1) * G]
                c = cv[hk * G:(hk + 1) * G]
                fs = lax.dot_general(
                    qg, k_all, (((1,), (1,)), ((), ())),
                    preferred_element_type=jnp.float32) * scale
                fsm = jnp.where(c > 0.0, fs, -1e30)
                m = jnp.max(fsm, axis=1, keepdims=True)
                p = c * jnp.exp(fsm - m)
                denom = jnp.sum(p, axis=1, keepdims=True)
                w = p / jnp.maximum(denom, 1e-30)
                out_ref[0, hk * G:(hk + 1) * G, 0, :] = lax.dot_general(
                    w, v_all, (((1,), (0,)), ((), ())),
                    preferred_element_type=jnp.float32)

        par = lax.rem(b, 2)

        @pl.when(par == 0)
        def _even():
            phase(0)

        @pl.when(par == 1)
        def _odd():
            phase(1)

    grid_spec = pltpu.PrefetchScalarGridSpec(
        num_scalar_prefetch=2,
        grid=(B,),
        in_specs=[
            pl.BlockSpec(memory_space=pl.ANY),
            pl.BlockSpec((1, H, 1, D), lambda b, *_: (b, 0, 0, 0)),
            pl.BlockSpec((1, H, _TOKENS), lambda b, *_: (b, 0, 0)),
        ],
        out_specs=pl.BlockSpec((1, H, 1, D), lambda b, *_: (b, 0, 0, 0)),
        scratch_shapes=[
            pltpu.VMEM((2, 2, Hkv, npages, page_size, D), jnp.float32),
            pltpu.SemaphoreType.DMA((2,)),
        ],
    )
    return pl.pallas_call(
        body,
        grid_spec=grid_spec,
        out_shape=jax.ShapeDtypeStruct((B, H, 1, D), jnp.float32),
        compiler_params=pltpu.CompilerParams(
            dimension_semantics=("arbitrary",)),
        interpret=interpret,
    )


def kernel(q, paged_kv_cache, kv_page_indptr, kv_page_indices, sparse_ind,
           sparse_nnz):
    B, H, _, D = q.shape
    _, _, Hkv, page_size, _ = paged_kv_cache.shape
    Lmax = sparse_ind.shape[2]
    npairs = B * H

    ind_flat = sparse_ind.reshape(npairs, Lmax)
    nnz_flat = sparse_nnz.reshape(npairs)

    counts = _sc_counts(npairs, Lmax)(ind_flat, nnz_flat)
    counts3 = counts.reshape(B, H, _TOKENS)

    return _tc_attention(B, H, Hkv, page_size, D)(
        kv_page_indptr, kv_page_indices, paged_kv_cache, q, counts3)


# speedup vs baseline: 66.9117x; 1.0418x over previous
import functools
import math

import jax
import jax.numpy as jnp
from jax import lax
from jax.experimental import pallas as pl
from jax.experimental.pallas import tpu as pltpu
from jax.experimental.pallas import tpu_sc as plsc

_NC = 2
_NS = 16
_LANES = 16

_TOKENS = 2048


def _sc_counts(npairs: int, lmax: int):
    nw = _NC * _NS
    per_w = npairs // nw
    span = per_w * _TOKENS
    region = span + 8
    nchunk = lmax // _LANES

    mesh = plsc.VectorSubcoreMesh(
        core_axis_name="c", subcore_axis_name="s",
        num_cores=_NC, num_subcores=_NS)

    @functools.partial(
        pl.kernel,
        out_type=jax.ShapeDtypeStruct((npairs * _TOKENS,), jnp.float32),
        mesh=mesh,
        scratch_types=[
            pltpu.VMEM((per_w, lmax), jnp.int32),
            pltpu.VMEM((per_w * lmax,), jnp.int32),
            pltpu.VMEM((per_w * lmax,), jnp.float32),
            pltpu.VMEM((_TOKENS,), jnp.float32),
            pltpu.VMEM((per_w * _LANES,), jnp.int32),
            pltpu.VMEM((per_w * _LANES,), jnp.int32),
            pltpu.VMEM_SHARED((_NS * region,), jnp.float32),
            pltpu.SemaphoreType.DMA,
            pltpu.SemaphoreType.DMA,
            pltpu.SemaphoreType.DMA,
        ],
        compiler_params=pltpu.CompilerParams(use_tc_tiling_on_sc=False),
    )
    def counts_kernel(ind_hbm, nnz_hbm, out_hbm, ind_all, idx_all, ones_v,
                      zeros_v, nnzidx_v, nnz16_all, acc_sh, sem_i, sem_n,
                      sem_z):
        cid = lax.axis_index("c")
        sid = lax.axis_index("s")
        wid = sid * _NC + cid
        base = sid * region
        row0 = wid * per_w

        cp_ind = pltpu.async_copy(ind_hbm.at[pl.ds(row0, per_w)], ind_all,
                                  sem_i)
        for p in range(per_w):
            nnzidx_v[pl.ds(p * _LANES, _LANES)] = lax.full(
                (_LANES,), row0 + p, jnp.int32)
        cp_nnz = pltpu.async_copy(nnz_hbm.at[nnzidx_v], nnz16_all, sem_n)
        for i in range(_TOKENS // _LANES):
            zeros_v[pl.ds(i * _LANES, _LANES)] = jnp.zeros((_LANES,),
                                                           jnp.float32)
        zcopies = []
        for p in range(per_w):
            zcopies.append(pltpu.async_copy(
                zeros_v, acc_sh.at[pl.ds(base + p * _TOKENS, _TOKENS)],
                sem_z))
        for i in range(per_w * lmax // _LANES):
            ones_v[pl.ds(i * _LANES, _LANES)] = jnp.full((_LANES,), 1.0,
                                                         jnp.float32)
        cp_ind.wait()
        cp_nnz.wait()
        for p in range(per_w):
            nnz_splat = nnz16_all[pl.ds(p * _LANES, _LANES)]
            pbase = base + p * _TOKENS
            for ci in range(nchunk):
                lvec = lax.iota(jnp.int32, _LANES) + ci * _LANES
                t = ind_all[p, pl.ds(ci * _LANES, _LANES)]
                ok = lvec < nnz_splat
                idx_all[pl.ds((p * nchunk + ci) * _LANES, _LANES)] = (
                    jnp.where(ok, t + pbase, base + span))
        for z in zcopies:
            z.wait()
        pltpu.sync_copy(ones_v, acc_sh.at[idx_all], add=True)
        pltpu.sync_copy(acc_sh.at[pl.ds(base, span)],
                        out_hbm.at[pl.ds(row0 * _TOKENS, span)])

    return counts_kernel


def _tc_attention(B, H, Hkv, page_size, D, interpret=False):
    G = H // Hkv
    npages = _TOKENS // page_size
    scale = 1.0 / math.sqrt(D)
    total = B * Hkv

    def body(indptr_s, pidx_s, cache, q_ref, c_ref, out_ref, kv_scr, sems):
        b = pl.program_id(0)

        def issue(dstbuf, bb):
            base = indptr_s[bb]

            def one(j, _):
                pid = pidx_s[base + j]
                pltpu.make_async_copy(
                    cache.at[pid],
                    kv_scr.at[dstbuf, :, :, j],
                    sems.at[dstbuf]).start()
                return 0

            lax.fori_loop(0, npages, one, 0)

        @pl.when(b == 0)
        def _prologue():
            issue(0, b)

        def phase(cur):
            @pl.when(b + 1 < B)
            def _():
                issue(1 - cur, b + 1)

            def wone(j, _):
                pltpu.make_async_copy(
                    cache.at[0],
                    kv_scr.at[cur, :, :, 0],
                    sems.at[cur]).wait()
                return 0

            lax.fori_loop(0, npages, wone, 0)

            qv = q_ref[0, :, 0, :]
            cv = c_ref[0]
            for hk in range(Hkv):
                k_all = kv_scr[cur, 0, hk].reshape(_TOKENS, D)
                v_all = kv_scr[cur, 1, hk].reshape(_TOKENS, D)
                qg = qv[hk * G:(hk + 1) * G]
                c = cv[hk * G:(hk + 1) * G]
                fs = lax.dot_general(
                    qg, k_all, (((1,), (1,)), ((), ())),
                    preferred_element_type=jnp.float32) * scale
                fsm = jnp.where(c > 0.0, fs, -1e30)
                m = jnp.max(fsm, axis=1, keepdims=True)
                p = c * jnp.exp(fsm - m)
                denom = jnp.sum(p, axis=1, keepdims=True)
                w = p / jnp.maximum(denom, 1e-30)
                out_ref[0, hk * G:(hk + 1) * G, 0, :] = lax.dot_general(
                    w, v_all, (((1,), (0,)), ((), ())),
                    preferred_element_type=jnp.float32)

        par = lax.rem(b, 2)

        @pl.when(par == 0)
        def _even():
            phase(0)

        @pl.when(par == 1)
        def _odd():
            phase(1)

    grid_spec = pltpu.PrefetchScalarGridSpec(
        num_scalar_prefetch=2,
        grid=(B,),
        in_specs=[
            pl.BlockSpec(memory_space=pl.ANY),
            pl.BlockSpec((1, H, 1, D), lambda b, *_: (b, 0, 0, 0)),
            pl.BlockSpec((1, H, _TOKENS), lambda b, *_: (b, 0, 0)),
        ],
        out_specs=pl.BlockSpec((1, H, 1, D), lambda b, *_: (b, 0, 0, 0)),
        scratch_shapes=[
            pltpu.VMEM((2, 2, Hkv, npages, page_size, D), jnp.float32),
            pltpu.SemaphoreType.DMA((2,)),
        ],
    )
    return pl.pallas_call(
        body,
        grid_spec=grid_spec,
        out_shape=jax.ShapeDtypeStruct((B, H, 1, D), jnp.float32),
        compiler_params=pltpu.CompilerParams(
            dimension_semantics=("arbitrary",)),
        interpret=interpret,
    )


def kernel(q, paged_kv_cache, kv_page_indptr, kv_page_indices, sparse_ind,
           sparse_nnz):
    B, H, _, D = q.shape
    _, _, Hkv, page_size, _ = paged_kv_cache.shape
    Lmax = sparse_ind.shape[2]
    npairs = B * H

    ind_flat = sparse_ind.reshape(npairs, Lmax)
    nnz_flat = sparse_nnz.reshape(npairs)

    counts = _sc_counts(npairs, Lmax)(ind_flat, nnz_flat)
    counts3 = counts.reshape(B, H, _TOKENS)

    return _tc_attention(B, H, Hkv, page_size, D)(
        kv_page_indptr, kv_page_indices, paged_kv_cache, q, counts3)


# speedup vs baseline: 71.6894x; 1.0714x over previous
import functools
import math

import jax
import jax.numpy as jnp
from jax import lax
from jax.experimental import pallas as pl
from jax.experimental.pallas import tpu as pltpu
from jax.experimental.pallas import tpu_sc as plsc

_NC = 2
_NS = 16
_LANES = 16

_TOKENS = 2048


def _sc_counts(npairs: int, lmax: int):
    nw = _NC * _NS
    per_w = npairs // nw
    span = per_w * _TOKENS
    region = span + 8
    nchunk = lmax // _LANES

    mesh = plsc.VectorSubcoreMesh(
        core_axis_name="c", subcore_axis_name="s",
        num_cores=_NC, num_subcores=_NS)

    @functools.partial(
        pl.kernel,
        out_type=jax.ShapeDtypeStruct((npairs * _TOKENS,), jnp.float32),
        mesh=mesh,
        scratch_types=[
            pltpu.VMEM((per_w, lmax), jnp.int32),
            pltpu.VMEM((per_w * lmax,), jnp.int32),
            pltpu.VMEM((per_w * lmax,), jnp.float32),
            pltpu.VMEM((_TOKENS,), jnp.float32),
            pltpu.VMEM((per_w * _LANES,), jnp.int32),
            pltpu.VMEM((per_w * _LANES,), jnp.int32),
            pltpu.VMEM_SHARED((_NS * region,), jnp.float32),
            pltpu.SemaphoreType.DMA,
            pltpu.SemaphoreType.DMA,
            pltpu.SemaphoreType.DMA,
        ],
        compiler_params=pltpu.CompilerParams(use_tc_tiling_on_sc=False),
    )
    def counts_kernel(ind_hbm, nnz_hbm, out_hbm, ind_all, idx_all, ones_v,
                      zeros_v, nnzidx_v, nnz16_all, acc_sh, sem_i, sem_n,
                      sem_z):
        cid = lax.axis_index("c")
        sid = lax.axis_index("s")
        wid = sid * _NC + cid
        base = sid * region
        row0 = wid * per_w

        cp_ind = pltpu.async_copy(ind_hbm.at[pl.ds(row0, per_w)], ind_all,
                                  sem_i)
        for p in range(per_w):
            nnzidx_v[pl.ds(p * _LANES, _LANES)] = lax.full(
                (_LANES,), row0 + p, jnp.int32)
        cp_nnz = pltpu.async_copy(nnz_hbm.at[nnzidx_v], nnz16_all, sem_n)
        for i in range(_TOKENS // _LANES):
            zeros_v[pl.ds(i * _LANES, _LANES)] = jnp.zeros((_LANES,),
                                                           jnp.float32)
        zcopies = []
        for p in range(per_w):
            zcopies.append(pltpu.async_copy(
                zeros_v, acc_sh.at[pl.ds(base + p * _TOKENS, _TOKENS)],
                sem_z))
        for i in range(per_w * lmax // _LANES):
            ones_v[pl.ds(i * _LANES, _LANES)] = jnp.full((_LANES,), 1.0,
                                                         jnp.float32)
        cp_ind.wait()
        cp_nnz.wait()
        for p in range(per_w):
            nnz_splat = nnz16_all[pl.ds(p * _LANES, _LANES)]
            pbase = base + p * _TOKENS
            for ci in range(nchunk):
                lvec = lax.iota(jnp.int32, _LANES) + ci * _LANES
                t = ind_all[p, pl.ds(ci * _LANES, _LANES)]
                ok = lvec < nnz_splat
                idx_all[pl.ds((p * nchunk + ci) * _LANES, _LANES)] = (
                    jnp.where(ok, t + pbase, base + span))
        for z in zcopies:
            z.wait()
        pltpu.sync_copy(ones_v, acc_sh.at[idx_all], add=True)
        pltpu.sync_copy(acc_sh.at[pl.ds(base, span)],
                        out_hbm.at[pl.ds(row0 * _TOKENS, span)])

    return counts_kernel


def _tc_scores(B, H, Hkv, page_size, D, interpret=False):
    G = H // Hkv
    npages = _TOKENS // page_size
    scale = 1.0 / math.sqrt(D)

    def body(indptr_s, pidx_s, cache, q_ref, fs_ref, k_scr, sems):
        b = pl.program_id(0)

        def issue(dstbuf, bb):
            base = indptr_s[bb]

            def one(j, _):
                pid = pidx_s[base + j]
                pltpu.make_async_copy(
                    cache.at[pid, 0],
                    k_scr.at[dstbuf, :, j],
                    sems.at[dstbuf]).start()
                return 0

            lax.fori_loop(0, npages, one, 0)

        @pl.when(b == 0)
        def _prologue():
            issue(0, b)

        def phase(cur):
            @pl.when(b + 1 < B)
            def _():
                issue(1 - cur, b + 1)

            def wone(j, _):
                pltpu.make_async_copy(
                    cache.at[0, 0], k_scr.at[cur, :, 0], sems.at[cur]).wait()
                return 0

            lax.fori_loop(0, npages, wone, 0)

            qv = q_ref[0, :, 0, :]
            for hk in range(Hkv):
                k_all = k_scr[cur, hk].reshape(_TOKENS, D)
                qg = qv[hk * G:(hk + 1) * G]
                fs_ref[0, hk * G:(hk + 1) * G, :] = lax.dot_general(
                    qg, k_all, (((1,), (1,)), ((), ())),
                    preferred_element_type=jnp.float32) * scale

        par = lax.rem(b, 2)

        @pl.when(par == 0)
        def _even():
            phase(0)

        @pl.when(par == 1)
        def _odd():
            phase(1)

    grid_spec = pltpu.PrefetchScalarGridSpec(
        num_scalar_prefetch=2,
        grid=(B,),
        in_specs=[
            pl.BlockSpec(memory_space=pl.ANY),
            pl.BlockSpec((1, H, 1, D), lambda b, *_: (b, 0, 0, 0)),
        ],
        out_specs=pl.BlockSpec((1, H, _TOKENS), lambda b, *_: (b, 0, 0)),
        scratch_shapes=[
            pltpu.VMEM((2, Hkv, npages, page_size, D), jnp.float32),
            pltpu.SemaphoreType.DMA((2,)),
        ],
    )
    return pl.pallas_call(
        body,
        grid_spec=grid_spec,
        out_shape=jax.ShapeDtypeStruct((B, H, _TOKENS), jnp.float32),
        compiler_params=pltpu.CompilerParams(
            dimension_semantics=("arbitrary",)),
        interpret=interpret,
    )


def _tc_pv(B, H, Hkv, page_size, D, interpret=False):
    G = H // Hkv
    npages = _TOKENS // page_size

    def body(indptr_s, pidx_s, cache, fs_in, c_ref, out_ref, v_scr, sems):
        b = pl.program_id(0)

        def issue(dstbuf, bb):
            base = indptr_s[bb]

            def one(j, _):
                pid = pidx_s[base + j]
                pltpu.make_async_copy(
                    cache.at[pid, 1],
                    v_scr.at[dstbuf, :, j],
                    sems.at[dstbuf]).start()
                return 0

            lax.fori_loop(0, npages, one, 0)

        @pl.when(b == 0)
        def _prologue():
            issue(0, b)

        def phase(cur):
            @pl.when(b + 1 < B)
            def _():
                issue(1 - cur, b + 1)

            def wone(j, _):
                pltpu.make_async_copy(
                    cache.at[0, 1], v_scr.at[cur, :, 0], sems.at[cur]).wait()
                return 0

            lax.fori_loop(0, npages, wone, 0)

            fsv = fs_in[0]
            cv = c_ref[0]
            for hk in range(Hkv):
                v_all = v_scr[cur, hk].reshape(_TOKENS, D)
                fs = fsv[hk * G:(hk + 1) * G]
                c = cv[hk * G:(hk + 1) * G]
                fsm = jnp.where(c > 0.0, fs, -1e30)
                m = jnp.max(fsm, axis=1, keepdims=True)
                p = c * jnp.exp(fsm - m)
                denom = jnp.sum(p, axis=1, keepdims=True)
                w = p / jnp.maximum(denom, 1e-30)
                out_ref[0, hk * G:(hk + 1) * G, 0, :] = lax.dot_general(
                    w, v_all, (((1,), (0,)), ((), ())),
                    preferred_element_type=jnp.float32)

        par = lax.rem(b, 2)

        @pl.when(par == 0)
        def _even():
            phase(0)

        @pl.when(par == 1)
        def _odd():
            phase(1)

    grid_spec = pltpu.PrefetchScalarGridSpec(
        num_scalar_prefetch=2,
        grid=(B,),
        in_specs=[
            pl.BlockSpec(memory_space=pl.ANY),
            pl.BlockSpec((1, H, _TOKENS), lambda b, *_: (b, 0, 0)),
            pl.BlockSpec((1, H, _TOKENS), lambda b, *_: (b, 0, 0)),
        ],
        out_specs=pl.BlockSpec((1, H, 1, D), lambda b, *_: (b, 0, 0, 0)),
        scratch_shapes=[
            pltpu.VMEM((2, Hkv, npages, page_size, D), jnp.float32),
            pltpu.SemaphoreType.DMA((2,)),
        ],
    )
    return pl.pallas_call(
        body,
        grid_spec=grid_spec,
        out_shape=jax.ShapeDtypeStruct((B, H, 1, D), jnp.float32),
        compiler_params=pltpu.CompilerParams(
            dimension_semantics=("arbitrary",)),
        interpret=interpret,
    )


def kernel(q, paged_kv_cache, kv_page_indptr, kv_page_indices, sparse_ind,
           sparse_nnz):
    B, H, _, D = q.shape
    _, _, Hkv, page_size, _ = paged_kv_cache.shape
    Lmax = sparse_ind.shape[2]
    npairs = B * H

    ind_flat = sparse_ind.reshape(npairs, Lmax)
    nnz_flat = sparse_nnz.reshape(npairs)

    counts = _sc_counts(npairs, Lmax)(ind_flat, nnz_flat)
    counts3 = counts.reshape(B, H, _TOKENS)

    fs = _tc_scores(B, H, Hkv, page_size, D)(
        kv_page_indptr, kv_page_indices, paged_kv_cache, q)
    return _tc_pv(B, H, Hkv, page_size, D)(
        kv_page_indptr, kv_page_indices, paged_kv_cache, fs, counts3)


# speedup vs baseline: 71.8587x; 1.0024x over previous
import functools
import math

import jax
import jax.numpy as jnp
from jax import lax
from jax.experimental import pallas as pl
from jax.experimental.pallas import tpu as pltpu
from jax.experimental.pallas import tpu_sc as plsc

_NC = 2
_NS = 16
_LANES = 16

_TOKENS = 2048


def _sc_counts(npairs: int, lmax: int):
    nw = _NC * _NS
    per_w = npairs // nw
    span = per_w * _TOKENS
    region = span + 8
    nchunk = lmax // _LANES

    mesh = plsc.VectorSubcoreMesh(
        core_axis_name="c", subcore_axis_name="s",
        num_cores=_NC, num_subcores=_NS)

    @functools.partial(
        pl.kernel,
        out_type=jax.ShapeDtypeStruct((npairs * _TOKENS,), jnp.float32),
        mesh=mesh,
        scratch_types=[
            pltpu.VMEM((per_w, lmax), jnp.int32),
            pltpu.VMEM((per_w * lmax,), jnp.int32),
            pltpu.VMEM((per_w * lmax,), jnp.float32),
            pltpu.VMEM((_TOKENS,), jnp.float32),
            pltpu.VMEM((per_w * _LANES,), jnp.int32),
            pltpu.VMEM((per_w * _LANES,), jnp.int32),
            pltpu.VMEM_SHARED((_NS * region,), jnp.float32),
            pltpu.SemaphoreType.DMA,
            pltpu.SemaphoreType.DMA,
            pltpu.SemaphoreType.DMA,
        ],
        compiler_params=pltpu.CompilerParams(use_tc_tiling_on_sc=False),
    )
    def counts_kernel(ind_hbm, nnz_hbm, out_hbm, ind_all, idx_all, ones_v,
                      zeros_v, nnzidx_v, nnz16_all, acc_sh, sem_i, sem_n,
                      sem_z):
        cid = lax.axis_index("c")
        sid = lax.axis_index("s")
        wid = sid * _NC + cid
        base = sid * region
        row0 = wid * per_w

        cp_ind = pltpu.async_copy(ind_hbm.at[pl.ds(row0, per_w)], ind_all,
                                  sem_i)
        for p in range(per_w):
            nnzidx_v[pl.ds(p * _LANES, _LANES)] = lax.full(
                (_LANES,), row0 + p, jnp.int32)
        cp_nnz = pltpu.async_copy(nnz_hbm.at[nnzidx_v], nnz16_all, sem_n)
        for i in range(_TOKENS // _LANES):
            zeros_v[pl.ds(i * _LANES, _LANES)] = jnp.zeros((_LANES,),
                                                           jnp.float32)
        zcopies = []
        for p in range(per_w):
            zcopies.append(pltpu.async_copy(
                zeros_v, acc_sh.at[pl.ds(base + p * _TOKENS, _TOKENS)],
                sem_z))
        for i in range(per_w * lmax // _LANES):
            ones_v[pl.ds(i * _LANES, _LANES)] = jnp.full((_LANES,), 1.0,
                                                         jnp.float32)
        cp_ind.wait()
        cp_nnz.wait()
        for p in range(per_w):
            nnz_splat = nnz16_all[pl.ds(p * _LANES, _LANES)]
            pbase = base + p * _TOKENS
            for ci in range(nchunk):
                lvec = lax.iota(jnp.int32, _LANES) + ci * _LANES
                t = ind_all[p, pl.ds(ci * _LANES, _LANES)]
                ok = lvec < nnz_splat
                idx_all[pl.ds((p * nchunk + ci) * _LANES, _LANES)] = (
                    jnp.where(ok, t + pbase, base + span))
        for z in zcopies:
            z.wait()
        pltpu.sync_copy(ones_v, acc_sh.at[idx_all], add=True)
        pltpu.sync_copy(acc_sh.at[pl.ds(base, span)],
                        out_hbm.at[pl.ds(row0 * _TOKENS, span)])

    return counts_kernel


def _tc_scores(B, H, Hkv, page_size, D, interpret=False):
    G = H // Hkv
    npages = _TOKENS // page_size
    scale = 1.0 / math.sqrt(D)

    def body(indptr_s, pidx_s, cache, q_ref, fs_ref, k_scr, sems):
        b = pl.program_id(0)

        def issue(dstbuf, bb):
            base = indptr_s[bb]

            def one(j, _):
                pid0 = pidx_s[base + 2 * j]
                pid1 = pidx_s[base + 2 * j + 1]
                pltpu.make_async_copy(
                    cache.at[pid0, 0],
                    k_scr.at[dstbuf, :, 2 * j],
                    sems.at[dstbuf, 0]).start()
                pltpu.make_async_copy(
                    cache.at[pid1, 0],
                    k_scr.at[dstbuf, :, 2 * j + 1],
                    sems.at[dstbuf, 1]).start()
                return 0

            lax.fori_loop(0, npages // 2, one, 0)

        @pl.when(b == 0)
        def _prologue():
            issue(0, b)

        def phase(cur):
            @pl.when(b + 1 < B)
            def _():
                issue(1 - cur, b + 1)

            def wone(j, _):
                pltpu.make_async_copy(
                    cache.at[0, 0], k_scr.at[cur, :, 0],
                    sems.at[cur, 0]).wait()
                pltpu.make_async_copy(
                    cache.at[0, 0], k_scr.at[cur, :, 0],
                    sems.at[cur, 1]).wait()
                return 0

            lax.fori_loop(0, npages // 2, wone, 0)

            qv = q_ref[0, :, 0, :]
            for hk in range(Hkv):
                k_all = k_scr[cur, hk].reshape(_TOKENS, D)
                qg = qv[hk * G:(hk + 1) * G]
                fs_ref[0, hk * G:(hk + 1) * G, :] = lax.dot_general(
                    qg, k_all, (((1,), (1,)), ((), ())),
                    preferred_element_type=jnp.float32) * scale

        par = lax.rem(b, 2)

        @pl.when(par == 0)
        def _even():
            phase(0)

        @pl.when(par == 1)
        def _odd():
            phase(1)

    grid_spec = pltpu.PrefetchScalarGridSpec(
        num_scalar_prefetch=2,
        grid=(B,),
        in_specs=[
            pl.BlockSpec(memory_space=pl.ANY),
            pl.BlockSpec((1, H, 1, D), lambda b, *_: (b, 0, 0, 0)),
        ],
        out_specs=pl.BlockSpec((1, H, _TOKENS), lambda b, *_: (b, 0, 0)),
        scratch_shapes=[
            pltpu.VMEM((2, Hkv, npages, page_size, D), jnp.float32),
            pltpu.SemaphoreType.DMA((2, 2)),
        ],
    )
    return pl.pallas_call(
        body,
        grid_spec=grid_spec,
        out_shape=jax.ShapeDtypeStruct((B, H, _TOKENS), jnp.float32),
        compiler_params=pltpu.CompilerParams(
            dimension_semantics=("arbitrary",)),
        interpret=interpret,
    )


def _tc_pv(B, H, Hkv, page_size, D, interpret=False):
    G = H // Hkv
    npages = _TOKENS // page_size

    def body(indptr_s, pidx_s, cache, fs_in, c_ref, out_ref, v_scr, sems):
        b = pl.program_id(0)

        def issue(dstbuf, bb):
            base = indptr_s[bb]

            def one(j, _):
                pid0 = pidx_s[base + 2 * j]
                pid1 = pidx_s[base + 2 * j + 1]
                pltpu.make_async_copy(
                    cache.at[pid0, 1],
                    v_scr.at[dstbuf, :, 2 * j],
                    sems.at[dstbuf, 0]).start()
                pltpu.make_async_copy(
                    cache.at[pid1, 1],
                    v_scr.at[dstbuf, :, 2 * j + 1],
                    sems.at[dstbuf, 1]).start()
                return 0

            lax.fori_loop(0, npages // 2, one, 0)

        @pl.when(b == 0)
        def _prologue():
            issue(0, b)

        def phase(cur):
            @pl.when(b + 1 < B)
            def _():
                issue(1 - cur, b + 1)

            def wone(j, _):
                pltpu.make_async_copy(
                    cache.at[0, 1], v_scr.at[cur, :, 0],
                    sems.at[cur, 0]).wait()
                pltpu.make_async_copy(
                    cache.at[0, 1], v_scr.at[cur, :, 0],
                    sems.at[cur, 1]).wait()
                return 0

            lax.fori_loop(0, npages // 2, wone, 0)

            fsv = fs_in[0]
            cv = c_ref[0]
            for hk in range(Hkv):
                v_all = v_scr[cur, hk].reshape(_TOKENS, D)
                fs = fsv[hk * G:(hk + 1) * G]
                c = cv[hk * G:(hk + 1) * G]
                fsm = jnp.where(c > 0.0, fs, -1e30)
                m = jnp.max(fsm, axis=1, keepdims=True)
                p = c * jnp.exp(fsm - m)
                denom = jnp.sum(p, axis=1, keepdims=True)
                w = p / jnp.maximum(denom, 1e-30)
                out_ref[0, hk * G:(hk + 1) * G, 0, :] = lax.dot_general(
                    w, v_all, (((1,), (0,)), ((), ())),
                    preferred_element_type=jnp.float32)

        par = lax.rem(b, 2)

        @pl.when(par == 0)
        def _even():
            phase(0)

        @pl.when(par == 1)
        def _odd():
            phase(1)

    grid_spec = pltpu.PrefetchScalarGridSpec(
        num_scalar_prefetch=2,
        grid=(B,),
        in_specs=[
            pl.BlockSpec(memory_space=pl.ANY),
            pl.BlockSpec((1, H, _TOKENS), lambda b, *_: (b, 0, 0)),
            pl.BlockSpec((1, H, _TOKENS), lambda b, *_: (b, 0, 0)),
        ],
        out_specs=pl.BlockSpec((1, H, 1, D), lambda b, *_: (b, 0, 0, 0)),
        scratch_shapes=[
            pltpu.VMEM((2, Hkv, npages, page_size, D), jnp.float32),
            pltpu.SemaphoreType.DMA((2, 2)),
        ],
    )
    return pl.pallas_call(
        body,
        grid_spec=grid_spec,
        out_shape=jax.ShapeDtypeStruct((B, H, 1, D), jnp.float32),
        compiler_params=pltpu.CompilerParams(
            dimension_semantics=("arbitrary",)),
        interpret=interpret,
    )


def kernel(q, paged_kv_cache, kv_page_indptr, kv_page_indices, sparse_ind,
           sparse_nnz):
    B, H, _, D = q.shape
    _, _, Hkv, page_size, _ = paged_kv_cache.shape
    Lmax = sparse_ind.shape[2]
    npairs = B * H

    ind_flat = sparse_ind.reshape(npairs, Lmax)
    nnz_flat = sparse_nnz.reshape(npairs)

    counts = _sc_counts(npairs, Lmax)(ind_flat, nnz_flat)
    counts3 = counts.reshape(B, H, _TOKENS)

    fs = _tc_scores(B, H, Hkv, page_size, D)(
        kv_page_indptr, kv_page_indices, paged_kv_cache, q)
    return _tc_pv(B, H, Hkv, page_size, D)(
        kv_page_indptr, kv_page_indices, paged_kv_cache, fs, counts3)
